# baseline (speedup 1.0000x reference)
import functools

import jax
import jax.numpy as jnp
from jax import lax
from jax.experimental import pallas as pl
from jax.experimental.pallas import tpu as pltpu
from jax.experimental.pallas import tpu_sc as plsc

_NC = 2
_NS = 16
_NW = _NC * _NS
_L = 16


def kernel(inputs, reverse_p_array):
    b, c, h, w = inputs.shape
    n = h * w
    ng = n // _L
    rows = b * c
    rows_per = rows // _NW
    b_per = rows_per // c
    ch = 64
    chunks_per_b = c // ch

    perm = reverse_p_array.astype(jnp.int32)

    mesh = plsc.VectorSubcoreMesh(core_axis_name="c", subcore_axis_name="s")

    @functools.partial(
        pl.kernel,
        mesh=mesh,
        compiler_params=pltpu.CompilerParams(
            needs_layout_passes=False, use_tc_tiling_on_sc=False),
        out_type=jax.ShapeDtypeStruct((b, c, h, w), jnp.float32),
        scratch_types=[
            pltpu.VMEM((n,), jnp.int32),
            pltpu.VMEM((ch, h, w), jnp.float32),
            pltpu.VMEM((ch, h, w), jnp.float32),
            pltpu.VMEM((ch, h, w), jnp.float32),
            pltpu.VMEM((ch, h, w), jnp.float32),
            pltpu.SemaphoreType.DMA,
            pltpu.SemaphoreType.DMA,
            pltpu.SemaphoreType.DMA,
            pltpu.SemaphoreType.DMA,
        ],
    )
    def run(x_hbm, perm_hbm, out_hbm, perm_v, in_v0, in_v1, out_v0, out_v1,
            sem_i0, sem_i1, sem_o0, sem_o1):
        wid = lax.axis_index("s") * _NC + lax.axis_index("c")
        pltpu.sync_copy(perm_hbm, perm_v)

        phs = []
        pws = []
        for g in range(ng):
            pv = perm_v[pl.ds(g * _L, _L)]
            phs.append(pv // w)
            pws.append(lax.rem(pv, w))

        b0 = wid * b_per
        in_bufs = (in_v0, in_v1)
        out_bufs = (out_v0, out_v1)
        sems_i = (sem_i0, sem_i1)
        sems_o = (sem_o0, sem_o1)
        n_chunks = b_per * chunks_per_b

        def in_copy(k, slot):
            bi = b0 + k // chunks_per_b
            c0 = lax.rem(k, chunks_per_b) * ch
            return pltpu.make_async_copy(
                x_hbm.at[bi, pl.ds(c0, ch)], in_bufs[slot], sems_i[slot])

        def out_copy(k, slot):
            bi = b0 + k // chunks_per_b
            c0 = lax.rem(k, chunks_per_b) * ch
            return pltpu.make_async_copy(
                out_bufs[slot], out_hbm.at[bi, pl.ds(c0, ch)], sems_o[slot])

        def permute_chunk(slot):
            src = in_bufs[slot]
            dst = out_bufs[slot]

            def row(r, carry):
                rv = jnp.full((_L,), r, dtype=jnp.int32)
                for g in range(ng):
                    v = plsc.load_gather(src, [rv, phs[g], pws[g]])
                    dst[r, g] = v
                return carry

            lax.fori_loop(0, ch, row, 0, unroll=2)

        in_copy(0, 0).start()

        def pair(p, carry):
            for slot in range(2):
                k = p * 2 + slot

                @pl.when(k + 1 < n_chunks)
                def _():
                    in_copy(k + 1, 1 - slot).start()

                in_copy(k, slot).wait()

                @pl.when(k >= 2)
                def _():
                    out_copy(k - 2, slot).wait()

                permute_chunk(slot)
                out_copy(k, slot).start()
            return carry

        lax.fori_loop(0, n_chunks // 2, pair, 0)
        out_copy(n_chunks - 2, 0).wait()
        out_copy(n_chunks - 1, 1).wait()

    return run(inputs, perm)


# speedup vs baseline: 19.6974x; 19.6974x over previous
import functools

import jax
import jax.numpy as jnp
from jax import lax
from jax.experimental import pallas as pl
from jax.experimental.pallas import tpu as pltpu
from jax.experimental.pallas import tpu_sc as plsc

_NC = 2
_NS = 16
_NW = _NC * _NS
_L = 16


def kernel(inputs, reverse_p_array):
    b, c, h, w = inputs.shape
    n = h * w
    nrows = b * n
    rows_per = nrows // _NW
    ch = 64
    n_chunks = rows_per // ch
    chunks_per_b = n // ch

    xt = jnp.transpose(inputs, (0, 2, 3, 1)).reshape(nrows, c)
    perm = reverse_p_array.astype(jnp.int32)

    mesh = plsc.VectorSubcoreMesh(core_axis_name="c", subcore_axis_name="s")

    @functools.partial(
        pl.kernel,
        mesh=mesh,
        compiler_params=pltpu.CompilerParams(
            needs_layout_passes=False, use_tc_tiling_on_sc=True),
        out_type=jax.ShapeDtypeStruct((nrows, c), jnp.float32),
        scratch_types=[
            pltpu.VMEM((n,), jnp.int32),
            pltpu.VMEM((ch,), jnp.int32),
            pltpu.VMEM((ch,), jnp.int32),
            pltpu.VMEM((ch, c), jnp.float32),
            pltpu.VMEM((ch, c), jnp.float32),
            pltpu.SemaphoreType.DMA,
            pltpu.SemaphoreType.DMA,
            pltpu.SemaphoreType.DMA,
            pltpu.SemaphoreType.DMA,
        ],
    )
    def run(xt_hbm, perm_hbm, out_hbm, perm_v, idx0, idx1, buf0, buf1,
            sem_g0, sem_g1, sem_s0, sem_s1):
        wid = lax.axis_index("s") * _NC + lax.axis_index("c")
        pltpu.sync_copy(perm_hbm, perm_v)

        base = wid * rows_per
        idxs = (idx0, idx1)
        bufs = (buf0, buf1)
        sems_g = (sem_g0, sem_g1)
        sems_s = (sem_s0, sem_s1)

        def build_idx(k, slot):
            o0 = base + k * ch
            bi = o0 // n
            q0 = lax.rem(o0, n)
            for v in range(ch // _L):
                pv = perm_v[pl.ds(q0 + v * _L, _L)]
                idxs[slot][pl.ds(v * _L, _L)] = pv + bi * n

        def gather(slot):
            return pltpu.make_async_copy(
                xt_hbm.at[idxs[slot]], bufs[slot], sems_g[slot])

        def scatter(k, slot):
            o0 = base + k * ch
            return pltpu.make_async_copy(
                bufs[slot], out_hbm.at[pl.ds(o0, ch)], sems_s[slot])

        build_idx(0, 0)
        gather(0).start()

        def pair(p, carry):
            for slot in range(2):
                k = p * 2 + slot

                @pl.when(k + 1 < n_chunks)
                def _():
                    @pl.when(k >= 1)
                    def _():
                        scatter(k - 1, 1 - slot).wait()

                    build_idx(k + 1, 1 - slot)
                    gather(1 - slot).start()

                gather(slot).wait()
                scatter(k, slot).start()
            return carry

        lax.fori_loop(0, n_chunks // 2, pair, 0)
        scatter(n_chunks - 2, 0).wait()
        scatter(n_chunks - 1, 1).wait()

    yt = run(xt, perm)
    return jnp.transpose(yt.reshape(b, h, w, c), (0, 3, 1, 2))


# speedup vs baseline: 19.7318x; 1.0018x over previous
import functools

import jax
import jax.numpy as jnp
from jax import lax
from jax.experimental import pallas as pl
from jax.experimental.pallas import tpu as pltpu
from jax.experimental.pallas import tpu_sc as plsc

_NC = 2
_NS = 16
_NW = _NC * _NS
_L = 16


def kernel(inputs, reverse_p_array):
    b, c, h, w = inputs.shape
    n = h * w
    nrows = b * n
    rows_per = nrows // _NW
    ch = 64
    n_chunks = rows_per // ch

    xt = jnp.transpose(inputs, (0, 2, 3, 1)).reshape(nrows, c)
    perm = reverse_p_array.astype(jnp.int32)

    mesh = plsc.VectorSubcoreMesh(core_axis_name="c", subcore_axis_name="s")

    @functools.partial(
        pl.kernel,
        mesh=mesh,
        compiler_params=pltpu.CompilerParams(
            needs_layout_passes=False, use_tc_tiling_on_sc=True),
        out_type=jax.ShapeDtypeStruct((nrows, c), jnp.float32),
        scratch_types=[
            pltpu.VMEM((n,), jnp.int32),
            pltpu.VMEM((ch,), jnp.int32),
            pltpu.VMEM((ch,), jnp.int32),
            pltpu.VMEM((ch, c), jnp.float32),
            pltpu.VMEM((ch, c), jnp.float32),
            pltpu.SemaphoreType.DMA,
            pltpu.SemaphoreType.DMA,
            pltpu.SemaphoreType.DMA,
            pltpu.SemaphoreType.DMA,
        ],
    )
    def run(xt_hbm, perm_hbm, out_hbm, perm_v, idx0, idx1, buf0, buf1,
            sem_g0, sem_g1, sem_s0, sem_s1):
        wid = lax.axis_index("s") * _NC + lax.axis_index("c")
        pltpu.sync_copy(perm_hbm, perm_v)

        base = wid * rows_per
        idxs = (idx0, idx1)
        bufs = (buf0, buf1)
        sems_g = (sem_g0, sem_g1)
        sems_s = (sem_s0, sem_s1)

        def build_idx(k, slot):
            o0 = base + k * ch
            bi = o0 // n
            q0 = lax.rem(o0, n)
            for v in range(ch // _L):
                pv = perm_v[pl.ds(q0 + v * _L, _L)]
                idxs[slot][pl.ds(v * _L, _L)] = pv + bi * n

        def gather(slot):
            return pltpu.make_async_copy(
                xt_hbm.at[idxs[slot]], bufs[slot], sems_g[slot])

        def scatter(k, slot):
            o0 = base + k * ch
            return pltpu.make_async_copy(
                bufs[slot], out_hbm.at[pl.ds(o0, ch)], sems_s[slot])

        build_idx(0, 0)
        gather(0).start()

        def pair(p, carry):
            for slot in range(2):
                k = p * 2 + slot

                @pl.when(k + 1 < n_chunks)
                def _():
                    @pl.when(k >= 1)
                    def _():
                        scatter(k - 1, 1 - slot).wait()

                    build_idx(k + 1, 1 - slot)
                    gather(1 - slot).start()

                gather(slot).wait()
                scatter(k, slot).start()
            return carry

        lax.fori_loop(0, n_chunks // 2, pair, 0)
        scatter(n_chunks - 2, 0).wait()
        scatter(n_chunks - 1, 1).wait()

    yt = run(xt, perm)
    return jnp.transpose(yt.reshape(b, h, w, c), (0, 3, 1, 2))
